# baseline (speedup 1.0000x reference)
import functools

import numpy as np
import jax
import jax.numpy as jnp
from jax import lax
from jax.experimental import pallas as pl
from jax.experimental.pallas import tpu as pltpu


_PAR1 = pltpu.CompilerParams(dimension_semantics=("parallel",))
_TB = (((1,), (1,)), ((), ()))


def _fused_kernel(x_ref, w1_ref, gt_ref, bt_ref, w2_ref,
                  wn_ref, b_ref, o_ref, *, T, H, W, fold, kt, k, pad):
    THW, C = x_ref.shape[1], x_ref.shape[2]
    HW = H * W
    nq = C // T

    x2 = x_ref[0].astype(jnp.float32)

    x3 = jnp.reshape(x2, (T, HW, C))
    pooledT = jnp.sum(x3, axis=1) * (1.0 / float(HW))
    ti = lax.broadcasted_iota(jnp.int32, (T, T), 0)
    ui = lax.broadcasted_iota(jnp.int32, (T, T), 1)
    accT = jnp.zeros((T, C), jnp.float32)
    for j in range(k):
        stj = jnp.where(ti == ui + (pad - j), 1.0, 0.0)
        sj = jnp.dot(stj, pooledT, preferred_element_type=jnp.float32)
        accT = accT + lax.dot_general(
            sj, w1_ref[j], _TB, preferred_element_type=jnp.float32)

    n = float(C * T)
    mu = jnp.sum(accT) * (1.0 / n)
    d = accT - mu
    var = jnp.sum(d * d) * (1.0 / n)
    yT = jnp.tanh(d * lax.rsqrt(var + 1e-5) * gt_ref[...] + bt_ref[...])
    actT = jax.nn.sigmoid(lax.dot_general(
        yT, w2_ref[...], _TB, preferred_element_type=jnp.float32))

    actB = jnp.reshape(jnp.broadcast_to(actT[:, None, :], (T, HW, C)),
                       (THW, C))
    m2 = x2 * actB

    fhat = jnp.concatenate(
        [jnp.concatenate([m2[t * HW:(t + 1) * HW, i * nq:(i + 1) * nq]
                          for t in range(T)], axis=1)
         for i in range(T)], axis=0)

    pr = lax.broadcasted_iota(jnp.int32, (C, C), 0)
    pc = lax.broadcasted_iota(jnp.int32, (C, C), 1)
    pi = jnp.where((pr % nq) * T + pr // nq == pc, 1.0, 0.0)
    fstd = jnp.dot(fhat, pi, preferred_element_type=jnp.float32)

    lane = lax.broadcasted_iota(jnp.int32, (THW, C), 1)
    sh = kt * HW
    zpad = jnp.zeros((sh, C), jnp.float32)
    up = jnp.concatenate([fstd[sh:], zpad], axis=0)
    dn = jnp.concatenate([zpad, fstd[:THW - sh]], axis=0)
    f1 = jnp.where(lane < fold, up, jnp.where(lane < 2 * fold, dn, fstd))

    row = lax.broadcasted_iota(jnp.int32, (THW, C), 0)
    hh = (row // W) % H
    ww = row % W
    acc = jnp.zeros((THW, C), jnp.float32)
    tap = 0
    for dh in (-1, 0, 1):
        for dw in (-1, 0, 1):
            s = dh * W + dw
            if s > 0:
                shf = jnp.concatenate(
                    [f1[s:], jnp.zeros((s, C), jnp.float32)], axis=0)
            elif s < 0:
                shf = jnp.concatenate(
                    [jnp.zeros((-s, C), jnp.float32), f1[:THW + s]], axis=0)
            else:
                shf = f1
            valid = ((hh + dh >= 0) & (hh + dh < H)
                     & (ww + dw >= 0) & (ww + dw < W))
            part = jnp.where(valid, shf, 0.0).astype(jnp.bfloat16)
            acc = acc + lax.dot_general(
                part, wn_ref[tap], _TB, preferred_element_type=jnp.float32)
            tap += 1

    y = jnp.tanh(acc + b_ref[...].astype(jnp.float32)) + fstd
    o_ref[0] = y.astype(o_ref.dtype)


def kernel(x, enh_w1, enh_gamma, enh_beta, enh_w2, w, b):
    B, T, C, H, W = x.shape
    HW, THW = H * W, T * H * W
    k = enh_w1.shape[2]

    w1n = jnp.transpose(enh_w1, (2, 0, 1))
    w2n = enh_w2[:, :, 0]
    gt = jnp.transpose(enh_gamma)
    bt = jnp.transpose(enh_beta)
    b2d = b.reshape(1, C)
    wn = (jnp.transpose(w, (2, 3, 0, 1)).reshape(9, C, C)
          .astype(jnp.bfloat16))

    x_cl = jnp.transpose(x, (0, 1, 3, 4, 2)).reshape(B, THW, C)

    body = functools.partial(_fused_kernel, T=T, H=H, W=W, fold=C // 3,
                             kt=int(np.floor(T * 0.25)), k=k,
                             pad=(k - 1) // 2)
    out_cl = pl.pallas_call(
        body,
        out_shape=jax.ShapeDtypeStruct((B, THW, C), x.dtype),
        grid=(B,),
        in_specs=[pl.BlockSpec((1, THW, C), lambda i: (i, 0, 0)),
                  pl.BlockSpec((k, C, C), lambda i: (0, 0, 0)),
                  pl.BlockSpec((T, C), lambda i: (0, 0)),
                  pl.BlockSpec((T, C), lambda i: (0, 0)),
                  pl.BlockSpec((C, C), lambda i: (0, 0)),
                  pl.BlockSpec((9, C, C), lambda i: (0, 0, 0)),
                  pl.BlockSpec((1, C), lambda i: (0, 0))],
        out_specs=pl.BlockSpec((1, THW, C), lambda i: (i, 0, 0)),
        compiler_params=_PAR1,
    )(x_cl, w1n, gt, bt, w2n, wn, b2d)

    return jnp.transpose(out_cl.reshape(B, T, H, W, C), (0, 1, 4, 2, 3))


# speedup vs baseline: 1.3282x; 1.3282x over previous
import functools

import numpy as np
import jax
import jax.numpy as jnp
from jax import lax
from jax.experimental import pallas as pl
from jax.experimental.pallas import tpu as pltpu


_PAR1 = pltpu.CompilerParams(dimension_semantics=("arbitrary",))
_TB = (((1,), (1,)), ((), ()))


def _fused_kernel(x_ref, w1_ref, gt_ref, bt_ref, w2_ref,
                  wn_ref, b_ref, o_ref, *, T, H, W, fold, kt, k, pad):
    nb, THW, C = x_ref.shape
    HW = H * W
    R = nb * THW
    nq = C // T

    x2 = jnp.reshape(x_ref[...].astype(jnp.float32), (R, C))

    x3 = jnp.reshape(x2, (nb * T, HW, C))
    pooledT = jnp.sum(x3, axis=1) * (1.0 / float(HW))
    ti = lax.broadcasted_iota(jnp.int32, (nb * T, nb * T), 0)
    ui = lax.broadcasted_iota(jnp.int32, (nb * T, nb * T), 1)
    accT = jnp.zeros((nb * T, C), jnp.float32)
    for j in range(k):
        stj = jnp.where((ti // T == ui // T) & (ti % T == ui % T + (pad - j)),
                        1.0, 0.0)
        sj = jnp.dot(stj, pooledT, preferred_element_type=jnp.float32)
        accT = accT + lax.dot_general(
            sj, w1_ref[j], _TB, preferred_element_type=jnp.float32)

    n = float(C * T)
    a3 = jnp.reshape(accT, (nb, T, C))
    mu = jnp.sum(a3, axis=(1, 2), keepdims=True) * (1.0 / n)
    d = a3 - mu
    var = jnp.sum(d * d, axis=(1, 2), keepdims=True) * (1.0 / n)
    yT = jnp.reshape(
        jnp.tanh(d * lax.rsqrt(var + 1e-5) * gt_ref[...][None]
                 + bt_ref[...][None]), (nb * T, C))
    actT = jax.nn.sigmoid(lax.dot_general(
        yT, w2_ref[...], _TB, preferred_element_type=jnp.float32))

    actB = jnp.reshape(jnp.broadcast_to(actT[:, None, :], (nb * T, HW, C)),
                       (R, C))
    m2 = x2 * actB

    fhat = jnp.concatenate(
        [jnp.concatenate([m2[b * THW + t * HW: b * THW + (t + 1) * HW,
                             i * nq:(i + 1) * nq]
                          for t in range(T)], axis=1)
         for b in range(nb) for i in range(T)], axis=0)

    pr = lax.broadcasted_iota(jnp.int32, (C, C), 0)
    pc = lax.broadcasted_iota(jnp.int32, (C, C), 1)
    pi = jnp.where((pr % nq) * T + pr // nq == pc, 1.0, 0.0)
    fstd = jnp.dot(fhat, pi, preferred_element_type=jnp.float32)

    row = lax.broadcasted_iota(jnp.int32, (R, C), 0)
    rb = row % THW
    lane = lax.broadcasted_iota(jnp.int32, (R, C), 1)
    sh = kt * HW
    zpad = jnp.zeros((sh, C), jnp.float32)
    up = jnp.where(rb < THW - sh,
                   jnp.concatenate([fstd[sh:], zpad], axis=0), 0.0)
    dn = jnp.where(rb >= sh,
                   jnp.concatenate([zpad, fstd[:R - sh]], axis=0), 0.0)
    f1 = jnp.where(lane < fold, up, jnp.where(lane < 2 * fold, dn, fstd))

    hh = (row // W) % H
    ww = row % W
    acc = jnp.zeros((R, C), jnp.float32)
    tap = 0
    for dh in (-1, 0, 1):
        for dw in (-1, 0, 1):
            s = dh * W + dw
            if s > 0:
                shf = jnp.concatenate(
                    [f1[s:], jnp.zeros((s, C), jnp.float32)], axis=0)
            elif s < 0:
                shf = jnp.concatenate(
                    [jnp.zeros((-s, C), jnp.float32), f1[:R + s]], axis=0)
            else:
                shf = f1
            valid = ((hh + dh >= 0) & (hh + dh < H)
                     & (ww + dw >= 0) & (ww + dw < W))
            part = jnp.where(valid, shf, 0.0)
            acc = acc + lax.dot_general(
                part, wn_ref[tap], _TB, preferred_element_type=jnp.float32)
            tap += 1

    y = jnp.tanh(acc + b_ref[...].astype(jnp.float32)) + fstd
    o_ref[...] = jnp.reshape(y, (nb, THW, C)).astype(o_ref.dtype)


def kernel(x, enh_w1, enh_gamma, enh_beta, enh_w2, w, b):
    B, T, C, H, W = x.shape
    HW, THW = H * W, T * H * W
    k = enh_w1.shape[2]
    nb = 2 if B % 2 == 0 else 1

    w1n = jnp.transpose(enh_w1, (2, 0, 1))
    w2n = enh_w2[:, :, 0]
    gt = jnp.transpose(enh_gamma)
    bt = jnp.transpose(enh_beta)
    b2d = b.reshape(1, C)
    wn = jnp.transpose(w, (2, 3, 0, 1)).reshape(9, C, C)

    x_cl = jnp.transpose(x, (0, 1, 3, 4, 2)).reshape(B, THW, C)

    body = functools.partial(_fused_kernel, T=T, H=H, W=W, fold=C // 3,
                             kt=int(np.floor(T * 0.25)), k=k,
                             pad=(k - 1) // 2)
    out_cl = pl.pallas_call(
        body,
        out_shape=jax.ShapeDtypeStruct((B, THW, C), x.dtype),
        grid=(B // nb,),
        in_specs=[pl.BlockSpec((nb, THW, C), lambda i: (i, 0, 0)),
                  pl.BlockSpec((k, C, C), lambda i: (0, 0, 0)),
                  pl.BlockSpec((T, C), lambda i: (0, 0)),
                  pl.BlockSpec((T, C), lambda i: (0, 0)),
                  pl.BlockSpec((C, C), lambda i: (0, 0)),
                  pl.BlockSpec((9, C, C), lambda i: (0, 0, 0)),
                  pl.BlockSpec((1, C), lambda i: (0, 0))],
        out_specs=pl.BlockSpec((nb, THW, C), lambda i: (i, 0, 0)),
        compiler_params=_PAR1,
    )(x_cl, w1n, gt, bt, w2n, wn, b2d)

    return jnp.transpose(out_cl.reshape(B, T, H, W, C), (0, 1, 4, 2, 3))
